# Initial kernel scaffold; baseline (speedup 1.0000x reference)
import jax
import jax.numpy as jnp
from jax.experimental import pallas as pl


def kernel(idx, chain_idx, emb_weight, emb_chain_weight):
    raise NotImplementedError("write your pallas kernel here")


# speedup vs baseline: 20.8408x; 20.8408x over previous
import functools
import jax
import jax.numpy as jnp
from jax import lax
from jax.experimental import pallas as pl
from jax.experimental.pallas import tpu as pltpu
from jax.experimental.pallas import tpu_sc as plsc

D = 128
NBIN = 65
NCBIN = 5
NTAB = NBIN * NCBIN
NC, NS, LN = 2, 16, 16
NW = NC * NS


def kernel(idx, chain_idx, emb_weight, emb_chain_weight):
    B, L = idx.shape
    rows = B * L * L
    pairs = B * L
    ppw = pairs // NW
    CH = 128
    NCHUNK = L // CH

    mesh = plsc.VectorSubcoreMesh(
        core_axis_name="c", subcore_axis_name="s", num_cores=NC, num_subcores=NS
    )

    @functools.partial(
        pl.kernel,
        out_type=(
            jax.ShapeDtypeStruct((rows, D), jnp.float32),
            jax.ShapeDtypeStruct((NC, NTAB, D), jnp.float32),
        ),
        mesh=mesh,
        compiler_params=pltpu.CompilerParams(needs_layout_passes=False),
        scratch_types=dict(
            idx_v=pltpu.VMEM((L,), jnp.int32),
            ch_v=pltpu.VMEM((L,), jnp.int32),
            cid_v=pltpu.VMEM((NCHUNK, CH), jnp.int32),
            row_v=pltpu.VMEM((CH, D), jnp.float32),
            emb_v=pltpu.VMEM((NBIN * D,), jnp.float32),
            cemb_v=pltpu.VMEM((NCBIN * D,), jnp.float32),
            tab_v=pltpu.VMEM((NTAB, D), jnp.float32),
            gsem=pltpu.SemaphoreType.DMA,
        ),
    )
    def run(idx_h, ch_h, emb_h, cemb_h, out_h, tab_h,
            idx_v, ch_v, cid_v, row_v, emb_v, cemb_v, tab_v, gsem):
        c = lax.axis_index("c")
        s = lax.axis_index("s")
        wid = s * NC + c

        @pl.when(s == 0)
        def _build_table():
            pltpu.sync_copy(emb_h, emb_v)
            pltpu.sync_copy(cemb_h, cemb_v)

            def build_m(m, carry):
                a = m // NCBIN
                cc = m % NCBIN
                for u in range(D // LN):
                    e = emb_v[pl.ds(a * D + u * LN, LN)]
                    ch = cemb_v[pl.ds(cc * D + u * LN, LN)]
                    tab_v[m, pl.ds(u * LN, LN)] = e + ch
                return carry

            lax.fori_loop(0, NTAB, build_m, 0)
            pltpu.sync_copy(tab_v, tab_h.at[c])

        plsc.subcore_barrier()

        b = wid // (NW // B)
        i0 = (wid * ppw) % L
        pltpu.sync_copy(idx_h.at[b], idx_v)
        pltpu.sync_copy(ch_h.at[b], ch_v)

        def pair_body(p, carry):
            i = i0 + p
            fi = jnp.full((LN,), i, jnp.int32)
            si = plsc.load_gather(idx_v, [fi])
            ci = plsc.load_gather(ch_v, [fi])
            for ck in range(NCHUNK):
                for u in range(CH // LN):
                    j0 = ck * CH + u * LN
                    dj = idx_v[pl.ds(j0, LN)]
                    cj = ch_v[pl.ds(j0, LN)]
                    ib = jnp.clip(dj - si + 32, 0, 64)
                    ic = jnp.clip(cj - ci + 2, 0, 4)
                    cid_v[ck, pl.ds(u * LN, LN)] = ib * NCBIN + ic
            row0 = pl.multiple_of((b * L + i) * L, L)
            for ck in range(NCHUNK):
                pltpu.async_copy(tab_h.at[c].at[cid_v.at[ck]], row_v, gsem).wait()
                pltpu.sync_copy(row_v, out_h.at[pl.ds(row0 + ck * CH, CH)])
            return carry

        lax.fori_loop(0, ppw, pair_body, 0)

    out_flat, _ = run(idx, chain_idx, emb_weight.reshape(-1),
                      emb_chain_weight.reshape(-1))
    return out_flat.reshape(B, L, L, D)


# speedup vs baseline: 20.9155x; 1.0036x over previous
import functools
import jax
import jax.numpy as jnp
from jax import lax
from jax.experimental import pallas as pl
from jax.experimental.pallas import tpu as pltpu
from jax.experimental.pallas import tpu_sc as plsc

D = 128
NBIN = 65
NCBIN = 5
NTAB = NBIN * NCBIN
NC, NS, LN = 2, 16, 16
NW = NC * NS


def kernel(idx, chain_idx, emb_weight, emb_chain_weight):
    B, L = idx.shape
    rows = B * L * L
    pairs = B * L
    ppw = pairs // NW
    CH = 128
    NCHUNK = L // CH

    mesh = plsc.VectorSubcoreMesh(
        core_axis_name="c", subcore_axis_name="s", num_cores=NC, num_subcores=NS
    )

    @functools.partial(
        pl.kernel,
        out_type=(
            jax.ShapeDtypeStruct((rows, D), jnp.float32),
            jax.ShapeDtypeStruct((NC, NTAB, D), jnp.float32),
        ),
        mesh=mesh,
        compiler_params=pltpu.CompilerParams(needs_layout_passes=False),
        scratch_types=dict(
            idx_v=pltpu.VMEM((L,), jnp.int32),
            ch_v=pltpu.VMEM((L,), jnp.int32),
            cid_v=pltpu.VMEM((NCHUNK, CH), jnp.int32),
            buf_a=pltpu.VMEM((2 * CH, D), jnp.float32),
            buf_b=pltpu.VMEM((2 * CH, D), jnp.float32),
            emb_v=pltpu.VMEM((NBIN * D,), jnp.float32),
            cemb_v=pltpu.VMEM((NCBIN * D,), jnp.float32),
            tab_v=pltpu.VMEM((NTAB, D), jnp.float32),
            gsem=pltpu.SemaphoreType.DMA,
            wsem_a=pltpu.SemaphoreType.DMA,
            wsem_b=pltpu.SemaphoreType.DMA,
        ),
    )
    def run(idx_h, ch_h, emb_h, cemb_h, out_h, tab_h,
            idx_v, ch_v, cid_v, buf_a, buf_b, emb_v, cemb_v, tab_v,
            gsem, wsem_a, wsem_b):
        c = lax.axis_index("c")
        s = lax.axis_index("s")
        wid = s * NC + c

        @pl.when(s == 0)
        def _build_table():
            pltpu.sync_copy(emb_h, emb_v)
            pltpu.sync_copy(cemb_h, cemb_v)

            def build_m(m, carry):
                a = m // NCBIN
                cc = m % NCBIN
                for u in range(D // LN):
                    e = emb_v[pl.ds(a * D + u * LN, LN)]
                    ch = cemb_v[pl.ds(cc * D + u * LN, LN)]
                    tab_v[m, pl.ds(u * LN, LN)] = e + ch
                return carry

            lax.fori_loop(0, NTAB, build_m, 0)
            pltpu.sync_copy(tab_v, tab_h.at[c])

        plsc.subcore_barrier()

        b = wid // (NW // B)
        i0 = (wid * ppw) % L
        pltpu.sync_copy(idx_h.at[b], idx_v)
        pltpu.sync_copy(ch_h.at[b], ch_v)

        def pair_body(p, carry):
            i = i0 + p
            fi = jnp.full((LN,), i, jnp.int32)
            si = plsc.load_gather(idx_v, [fi])
            ci = plsc.load_gather(ch_v, [fi])
            for ck in range(NCHUNK):
                for u in range(CH // LN):
                    j0 = ck * CH + u * LN
                    dj = idx_v[pl.ds(j0, LN)]
                    cj = ch_v[pl.ds(j0, LN)]
                    ib = jnp.clip(dj - si + 32, 0, 64)
                    ic = jnp.clip(cj - ci + 2, 0, 4)
                    cid_v[ck, pl.ds(u * LN, LN)] = ib * NCBIN + ic
            row0 = pl.multiple_of((b * L + i) * L, L)
            HCH = 2 * CH

            @pl.when(p > 0)
            def _wait_a():
                pltpu.make_async_copy(buf_a, out_h.at[pl.ds(row0, HCH)],
                                      wsem_a).wait()

            ga0 = pltpu.async_copy(tab_h.at[c].at[cid_v.at[0]],
                                   buf_a.at[pl.ds(0, CH)], gsem)
            ga1 = pltpu.async_copy(tab_h.at[c].at[cid_v.at[1]],
                                   buf_a.at[pl.ds(CH, CH)], gsem)

            @pl.when(p > 0)
            def _wait_b():
                pltpu.make_async_copy(buf_b, out_h.at[pl.ds(row0, HCH)],
                                      wsem_b).wait()

            gb0 = pltpu.async_copy(tab_h.at[c].at[cid_v.at[2]],
                                   buf_b.at[pl.ds(0, CH)], gsem)
            gb1 = pltpu.async_copy(tab_h.at[c].at[cid_v.at[3]],
                                   buf_b.at[pl.ds(CH, CH)], gsem)
            ga0.wait()
            ga1.wait()
            pltpu.async_copy(buf_a, out_h.at[pl.ds(row0, HCH)], wsem_a)
            gb0.wait()
            gb1.wait()
            pltpu.async_copy(buf_b, out_h.at[pl.ds(row0 + HCH, HCH)], wsem_b)
            return carry

        lax.fori_loop(0, ppw, pair_body, 0)
        last0 = pl.multiple_of((b * L + i0 + ppw - 1) * L, L)
        pltpu.make_async_copy(buf_a, out_h.at[pl.ds(last0, 2 * CH)],
                              wsem_a).wait()
        pltpu.make_async_copy(buf_b, out_h.at[pl.ds(last0 + 2 * CH, 2 * CH)],
                              wsem_b).wait()

    out_flat, _ = run(idx, chain_idx, emb_weight.reshape(-1),
                      emb_chain_weight.reshape(-1))
    return out_flat.reshape(B, L, L, D)


# speedup vs baseline: 396.2714x; 18.9463x over previous
import functools
import jax
import jax.numpy as jnp
from jax import lax
from jax.experimental import pallas as pl
from jax.experimental.pallas import tpu as pltpu
from jax.experimental.pallas import tpu_sc as plsc

D = 128
NBIN = 65
NCBIN = 5
NTAB = NBIN * NCBIN
NC, NS, LN = 2, 16, 16
NW = NC * NS


def kernel(idx, chain_idx, emb_weight, emb_chain_weight):
    B, L = idx.shape
    rows = B * L * L
    pairs = B * L
    ppw = pairs // NW
    CH = 128
    NCHUNK = L // CH

    mesh = plsc.VectorSubcoreMesh(
        core_axis_name="c", subcore_axis_name="s", num_cores=NC, num_subcores=NS
    )

    @functools.partial(
        pl.kernel,
        out_type=jax.ShapeDtypeStruct((rows, D), jnp.float32),
        mesh=mesh,
        compiler_params=pltpu.CompilerParams(needs_layout_passes=False),
        scratch_types=dict(
            idx_v=pltpu.VMEM((L,), jnp.int32),
            ch_v=pltpu.VMEM((L,), jnp.int32),
            cid_v=pltpu.VMEM((NCHUNK, CH), jnp.int32),
            buf_a=pltpu.VMEM((2 * CH, D), jnp.float32),
            buf_b=pltpu.VMEM((2 * CH, D), jnp.float32),
            emb_v=pltpu.VMEM((NBIN * D,), jnp.float32),
            cemb_v=pltpu.VMEM((NCBIN * D,), jnp.float32),
            tab_v=pltpu.VMEM((NTAB, D), jnp.float32),
            tab_s=pltpu.VMEM_SHARED((NTAB, D), jnp.float32),
            gsem=pltpu.SemaphoreType.DMA,
            wsem_a=pltpu.SemaphoreType.DMA,
            wsem_b=pltpu.SemaphoreType.DMA,
        ),
    )
    def run(idx_h, ch_h, emb_h, cemb_h, out_h,
            idx_v, ch_v, cid_v, buf_a, buf_b, emb_v, cemb_v, tab_v, tab_s,
            gsem, wsem_a, wsem_b):
        c = lax.axis_index("c")
        s = lax.axis_index("s")
        wid = s * NC + c

        @pl.when(s == 0)
        def _build_table():
            pltpu.sync_copy(emb_h, emb_v)
            pltpu.sync_copy(cemb_h, cemb_v)

            def build_m(m, carry):
                a = m // NCBIN
                cc = m % NCBIN
                for u in range(D // LN):
                    e = emb_v[pl.ds(a * D + u * LN, LN)]
                    ch = cemb_v[pl.ds(cc * D + u * LN, LN)]
                    tab_v[m, pl.ds(u * LN, LN)] = e + ch
                return carry

            lax.fori_loop(0, NTAB, build_m, 0)
            pltpu.sync_copy(tab_v, tab_s)

        plsc.subcore_barrier()

        b = wid // (NW // B)
        i0 = (wid * ppw) % L
        pltpu.sync_copy(idx_h.at[b], idx_v)
        pltpu.sync_copy(ch_h.at[b], ch_v)

        def pair_body(p, carry):
            i = i0 + p
            fi = jnp.full((LN,), i, jnp.int32)
            si = plsc.load_gather(idx_v, [fi])
            ci = plsc.load_gather(ch_v, [fi])
            for ck in range(NCHUNK):
                for u in range(CH // LN):
                    j0 = ck * CH + u * LN
                    dj = idx_v[pl.ds(j0, LN)]
                    cj = ch_v[pl.ds(j0, LN)]
                    ib = jnp.clip(dj - si + 32, 0, 64)
                    ic = jnp.clip(cj - ci + 2, 0, 4)
                    cid_v[ck, pl.ds(u * LN, LN)] = ib * NCBIN + ic
            row0 = pl.multiple_of((b * L + i) * L, L)
            HCH = 2 * CH

            @pl.when(p > 0)
            def _wait_a():
                pltpu.make_async_copy(buf_a, out_h.at[pl.ds(row0, HCH)],
                                      wsem_a).wait()

            ga0 = pltpu.async_copy(tab_s.at[cid_v.at[0]],
                                   buf_a.at[pl.ds(0, CH)], gsem)
            ga1 = pltpu.async_copy(tab_s.at[cid_v.at[1]],
                                   buf_a.at[pl.ds(CH, CH)], gsem)

            @pl.when(p > 0)
            def _wait_b():
                pltpu.make_async_copy(buf_b, out_h.at[pl.ds(row0, HCH)],
                                      wsem_b).wait()

            gb0 = pltpu.async_copy(tab_s.at[cid_v.at[2]],
                                   buf_b.at[pl.ds(0, CH)], gsem)
            gb1 = pltpu.async_copy(tab_s.at[cid_v.at[3]],
                                   buf_b.at[pl.ds(CH, CH)], gsem)
            ga0.wait()
            ga1.wait()
            pltpu.async_copy(buf_a, out_h.at[pl.ds(row0, HCH)], wsem_a)
            gb0.wait()
            gb1.wait()
            pltpu.async_copy(buf_b, out_h.at[pl.ds(row0 + HCH, HCH)], wsem_b)
            return carry

        lax.fori_loop(0, ppw, pair_body, 0)
        last0 = pl.multiple_of((b * L + i0 + ppw - 1) * L, L)
        pltpu.make_async_copy(buf_a, out_h.at[pl.ds(last0, 2 * CH)],
                              wsem_a).wait()
        pltpu.make_async_copy(buf_b, out_h.at[pl.ds(last0 + 2 * CH, 2 * CH)],
                              wsem_b).wait()

    out_flat = run(idx, chain_idx, emb_weight.reshape(-1),
                   emb_chain_weight.reshape(-1))
    return out_flat.reshape(B, L, L, D)


# speedup vs baseline: 429.7817x; 1.0846x over previous
import functools
import jax
import jax.numpy as jnp
from jax import lax
from jax.experimental import pallas as pl
from jax.experimental.pallas import tpu as pltpu
from jax.experimental.pallas import tpu_sc as plsc

D = 128
NBIN = 65
NCBIN = 5
NTAB = NBIN * NCBIN
NC, NS, LN = 2, 16, 16
NW = NC * NS


def kernel(idx, chain_idx, emb_weight, emb_chain_weight):
    B, L = idx.shape
    rows = B * L * L
    pairs = B * L
    ppw = pairs // NW
    RPT = -(-NTAB // NS)
    CH = 128
    NCHUNK = L // CH

    mesh = plsc.VectorSubcoreMesh(
        core_axis_name="c", subcore_axis_name="s", num_cores=NC, num_subcores=NS
    )

    @functools.partial(
        pl.kernel,
        out_type=jax.ShapeDtypeStruct((rows, D), jnp.float32),
        mesh=mesh,
        compiler_params=pltpu.CompilerParams(needs_layout_passes=False),
        scratch_types=dict(
            idx_v=pltpu.VMEM((L,), jnp.int32),
            ch_v=pltpu.VMEM((L,), jnp.int32),
            cid_v=pltpu.VMEM((NCHUNK, CH), jnp.int32),
            buf_a=pltpu.VMEM((2 * CH, D), jnp.float32),
            buf_b=pltpu.VMEM((2 * CH, D), jnp.float32),
            emb_v=pltpu.VMEM((NBIN * D,), jnp.float32),
            cemb_v=pltpu.VMEM((NCBIN * D,), jnp.float32),
            tab_v=pltpu.VMEM((RPT, D), jnp.float32),
            tab_s=pltpu.VMEM_SHARED((NS * RPT, D), jnp.float32),
            gsem=pltpu.SemaphoreType.DMA,
            wsem_a=pltpu.SemaphoreType.DMA,
            wsem_b=pltpu.SemaphoreType.DMA,
        ),
    )
    def run(idx_h, ch_h, emb_h, cemb_h, out_h,
            idx_v, ch_v, cid_v, buf_a, buf_b, emb_v, cemb_v, tab_v, tab_s,
            gsem, wsem_a, wsem_b):
        c = lax.axis_index("c")
        s = lax.axis_index("s")
        wid = s * NC + c

        b = wid // (NW // B)
        i0 = (wid * ppw) % L
        pltpu.sync_copy(idx_h.at[b], idx_v)
        pltpu.sync_copy(ch_h.at[b], ch_v)
        pltpu.sync_copy(emb_h, emb_v)
        pltpu.sync_copy(cemb_h, cemb_v)

        m0 = s * RPT

        def build_k(k, carry):
            m = jnp.minimum(m0 + k, NTAB - 1)
            a = m // NCBIN
            cc = m % NCBIN
            for u in range(D // LN):
                e = emb_v[pl.ds(a * D + u * LN, LN)]
                ch = cemb_v[pl.ds(cc * D + u * LN, LN)]
                tab_v[k, pl.ds(u * LN, LN)] = e + ch
            return carry

        lax.fori_loop(0, RPT, build_k, 0)
        pltpu.sync_copy(tab_v, tab_s.at[pl.ds(m0, RPT)])
        plsc.subcore_barrier()

        def pair_body(p, carry):
            i = i0 + p
            fi = jnp.full((LN,), i, jnp.int32)
            si = plsc.load_gather(idx_v, [fi])
            ci = plsc.load_gather(ch_v, [fi])
            for ck in range(NCHUNK):
                for u in range(CH // LN):
                    j0 = ck * CH + u * LN
                    dj = idx_v[pl.ds(j0, LN)]
                    cj = ch_v[pl.ds(j0, LN)]
                    ib = jnp.clip(dj - si + 32, 0, 64)
                    ic = jnp.clip(cj - ci + 2, 0, 4)
                    cid_v[ck, pl.ds(u * LN, LN)] = ib * NCBIN + ic
            row0 = pl.multiple_of((b * L + i) * L, L)
            HCH = 2 * CH

            @pl.when(p > 0)
            def _wait_a():
                pltpu.make_async_copy(buf_a, out_h.at[pl.ds(row0, HCH)],
                                      wsem_a).wait()

            ga0 = pltpu.async_copy(tab_s.at[cid_v.at[0]],
                                   buf_a.at[pl.ds(0, CH)], gsem)
            ga1 = pltpu.async_copy(tab_s.at[cid_v.at[1]],
                                   buf_a.at[pl.ds(CH, CH)], gsem)

            @pl.when(p > 0)
            def _wait_b():
                pltpu.make_async_copy(buf_b, out_h.at[pl.ds(row0, HCH)],
                                      wsem_b).wait()

            gb0 = pltpu.async_copy(tab_s.at[cid_v.at[2]],
                                   buf_b.at[pl.ds(0, CH)], gsem)
            gb1 = pltpu.async_copy(tab_s.at[cid_v.at[3]],
                                   buf_b.at[pl.ds(CH, CH)], gsem)
            ga0.wait()
            ga1.wait()
            pltpu.async_copy(buf_a, out_h.at[pl.ds(row0, HCH)], wsem_a)
            gb0.wait()
            gb1.wait()
            pltpu.async_copy(buf_b, out_h.at[pl.ds(row0 + HCH, HCH)], wsem_b)
            return carry

        lax.fori_loop(0, ppw, pair_body, 0)
        last0 = pl.multiple_of((b * L + i0 + ppw - 1) * L, L)
        pltpu.make_async_copy(buf_a, out_h.at[pl.ds(last0, 2 * CH)],
                              wsem_a).wait()
        pltpu.make_async_copy(buf_b, out_h.at[pl.ds(last0 + 2 * CH, 2 * CH)],
                              wsem_b).wait()

    out_flat = run(idx, chain_idx, emb_weight.reshape(-1),
                   emb_chain_weight.reshape(-1))
    return out_flat.reshape(B, L, L, D)
